# baseline (device time: 74265 ns/iter reference)
import jax
import jax.numpy as jnp
from jax import lax
from jax.experimental import pallas as pl
from jax.experimental.pallas import tpu as pltpu

N_DEV = 16


def kernel(x, w_mat, scale_x, scale_w):
    m_per, k = x.shape
    _, n = w_mat.shape
    n_per = n // N_DEV

    def body(x_ref, w_ref, sx_ref, sw_ref, out_ref, xq_scratch, y_scratch,
             rstage, send_sems, recv_sem):
        jj = pl.program_id(0)
        my_i = lax.axis_index("i")
        tgt = lax.rem(my_i + jj, N_DEV)

        @pl.when(jj == 0)
        def _():
            xq_scratch[...] = x_ref[...].astype(jnp.float8_e4m3fn)

        scale = sx_ref[0] * sw_ref[0]
        wq = w_ref[...].astype(jnp.float8_e5m2)
        y = jnp.dot(xq_scratch[...], wq, preferred_element_type=jnp.float32) * scale

        @pl.when(jj == 0)
        def _():
            rstage[my_i] = y.astype(jnp.bfloat16)

        @pl.when(jj > 0)
        def _():
            slot = jj - 1
            y_scratch[slot] = y.astype(jnp.bfloat16)
            rdma = pltpu.make_async_remote_copy(
                src_ref=y_scratch.at[slot],
                dst_ref=rstage.at[my_i],
                send_sem=send_sems.at[slot],
                recv_sem=recv_sem,
                device_id=(tgt,),
                device_id_type=pl.DeviceIdType.MESH,
            )
            rdma.start()

        @pl.when(jj == N_DEV - 1)
        def _():
            for s in range(N_DEV - 1):
                dummy = pltpu.make_async_remote_copy(
                    src_ref=y_scratch.at[s],
                    dst_ref=y_scratch.at[s],
                    send_sem=send_sems.at[s],
                    recv_sem=recv_sem,
                    device_id=(my_i,),
                    device_id_type=pl.DeviceIdType.MESH,
                )
                dummy.wait_send()
                dummy.wait_recv()
            out_ref[...] = rstage[...].reshape(N_DEV * m_per, n_per).astype(
                jnp.float32
            )

    grid = (N_DEV,)
    return pl.pallas_call(
        body,
        grid=grid,
        in_specs=[
            pl.BlockSpec((m_per, k), lambda jj: (0, 0)),
            pl.BlockSpec(
                (k, n_per),
                lambda jj: (0, lax.rem(lax.axis_index("i") + jj, N_DEV)),
            ),
            pl.BlockSpec(memory_space=pltpu.SMEM),
            pl.BlockSpec(memory_space=pltpu.SMEM),
        ],
        out_specs=pl.BlockSpec((N_DEV * m_per, n_per), lambda jj: (0, 0)),
        out_shape=jax.ShapeDtypeStruct((N_DEV * m_per, n_per), jnp.float32),
        scratch_shapes=[
            pltpu.VMEM((m_per, k), jnp.float8_e4m3fn),
            pltpu.VMEM((N_DEV - 1, m_per, n_per), jnp.bfloat16),
            pltpu.VMEM((N_DEV, m_per, n_per), jnp.bfloat16),
            pltpu.SemaphoreType.DMA((N_DEV - 1,)),
            pltpu.SemaphoreType.DMA,
        ],
        compiler_params=pltpu.CompilerParams(
            dimension_semantics=("arbitrary",),
        ),
    )(x, w_mat, scale_x, scale_w)


# device time: 56386 ns/iter; 1.3171x vs baseline; 1.3171x over previous
import jax
import jax.numpy as jnp
from jax import lax
from jax.experimental import pallas as pl
from jax.experimental.pallas import tpu as pltpu

N_DEV = 16


def kernel(x, w_mat, scale_x, scale_w):
    m_per, k = x.shape
    _, n = w_mat.shape
    n_per = n // N_DEV

    def _xor_off(jj):
        return jnp.where(
            jj == 0,
            0,
            jnp.where(jj <= 8, jj + 7, jnp.where(jj <= 12, jj - 5, jj - 12)),
        )

    def body(x_ref, w_ref, sx_ref, sw_ref, out_ref, xq_scratch, y_scratch,
             rstage, send_sems, recv_sem):
        jj = pl.program_id(0)
        my_i = lax.axis_index("i")
        tgt = lax.bitwise_xor(my_i, _xor_off(jj))

        @pl.when(jj == 0)
        def _():
            xq_scratch[...] = x_ref[...].astype(jnp.float8_e4m3fn)

        scale = sx_ref[0] * sw_ref[0]
        wq = w_ref[...].astype(jnp.float8_e5m2)
        y = jnp.dot(xq_scratch[...], wq, preferred_element_type=jnp.float32) * scale

        @pl.when(jj == 0)
        def _():
            rstage[my_i] = y.astype(jnp.bfloat16)

        @pl.when(jj > 0)
        def _():
            slot = jj - 1
            y_scratch[slot] = y.astype(jnp.bfloat16)
            rdma = pltpu.make_async_remote_copy(
                src_ref=y_scratch.at[slot],
                dst_ref=rstage.at[my_i],
                send_sem=send_sems.at[slot],
                recv_sem=recv_sem,
                device_id=(tgt,),
                device_id_type=pl.DeviceIdType.MESH,
            )
            rdma.start()

        @pl.when(jj == N_DEV - 1)
        def _():
            for s in range(N_DEV - 1):
                dummy = pltpu.make_async_remote_copy(
                    src_ref=y_scratch.at[s],
                    dst_ref=y_scratch.at[s],
                    send_sem=send_sems.at[s],
                    recv_sem=recv_sem,
                    device_id=(my_i,),
                    device_id_type=pl.DeviceIdType.MESH,
                )
                dummy.wait_send()
                dummy.wait_recv()
            out_ref[...] = rstage[...].reshape(N_DEV * m_per, n_per).astype(
                jnp.float32
            )

    grid = (N_DEV,)
    return pl.pallas_call(
        body,
        grid=grid,
        in_specs=[
            pl.BlockSpec((m_per, k), lambda jj: (0, 0)),
            pl.BlockSpec(
                (k, n_per),
                lambda jj: (
                    0,
                    lax.bitwise_xor(lax.axis_index("i"), _xor_off(jj)),
                ),
            ),
            pl.BlockSpec(memory_space=pltpu.SMEM),
            pl.BlockSpec(memory_space=pltpu.SMEM),
        ],
        out_specs=pl.BlockSpec((N_DEV * m_per, n_per), lambda jj: (0, 0)),
        out_shape=jax.ShapeDtypeStruct((N_DEV * m_per, n_per), jnp.float32),
        scratch_shapes=[
            pltpu.VMEM((m_per, k), jnp.float8_e4m3fn),
            pltpu.VMEM((N_DEV - 1, m_per, n_per), jnp.bfloat16),
            pltpu.VMEM((N_DEV, m_per, n_per), jnp.bfloat16),
            pltpu.SemaphoreType.DMA((N_DEV - 1,)),
            pltpu.SemaphoreType.DMA,
        ],
        compiler_params=pltpu.CompilerParams(
            dimension_semantics=("arbitrary",),
        ),
    )(x, w_mat, scale_x, scale_w)
